# Initial kernel scaffold; baseline (speedup 1.0000x reference)
import jax
import jax.numpy as jnp
from jax.experimental import pallas as pl


def kernel(x, w1, g1, be1, w2, g2, be2):
    raise NotImplementedError("write your pallas kernel here")


# speedup vs baseline: 1.0881x; 1.0881x over previous
import functools

import jax
import jax.numpy as jnp
from jax.experimental import pallas as pl
from jax.experimental.pallas import tpu as pltpu

_EPS = 1e-5
_VMEM_LIMIT = 48 * 2**20


def _emit_cols(a, col_ref, dims):
    c, d2, h2, w2 = dims
    m = d2 * h2 * w2
    sh, sd = w2, h2 * w2
    pad = sd + sh + 1
    zc = jnp.zeros((c, pad), jnp.float32)
    ae = jnp.concatenate([zc, a, zc], axis=1)
    lane = jax.lax.broadcasted_iota(jnp.int32, (1, m), 1)
    ww = jax.lax.rem(lane, w2)
    hh = jax.lax.rem(lane // w2, h2)
    t = 0
    for kd in range(3):
        for kh in range(3):
            for kw in range(3):
                delta = (kd - 1) * sd + (kh - 1) * sh + (kw - 1)
                sl = jax.lax.slice(ae, (0, pad + delta), (c, pad + delta + m))
                conds = []
                if kh == 0:
                    conds.append(hh >= 1)
                elif kh == 2:
                    conds.append(hh < h2 - 1)
                if kw == 0:
                    conds.append(ww >= 1)
                elif kw == 2:
                    conds.append(ww < w2 - 1)
                if conds:
                    mask = functools.reduce(jnp.logical_and, conds)
                    sl = jnp.where(mask, sl, 0.0)
                col_ref[t * c:(t + 1) * c, :] = sl.astype(col_ref.dtype)
                t += 1


def _conv_tail(w_ref, y_ref, sum_ref, ssq_ref, col_ref):
    y = jnp.dot(w_ref[...], col_ref[...], preferred_element_type=jnp.float32)
    y_ref[0] = y
    sum_ref[0] = jnp.sum(y, axis=1, keepdims=True)
    ssq_ref[0] = jnp.sum(y * y, axis=1, keepdims=True)


def _pool_conv_kernel(views_ref, w_ref, y_ref, sum_ref, ssq_ref, col_ref,
                      *, dims):
    pooled = jnp.max(views_ref[...], axis=0)[0]
    _emit_cols(pooled, col_ref, dims)
    _conv_tail(w_ref, y_ref, sum_ref, ssq_ref, col_ref)


def _bn_conv_kernel(scale_ref, shift_ref, w_ref, x_ref, y_ref, sum_ref,
                    ssq_ref, col_ref, *, dims):
    h = jnp.maximum(x_ref[0] * scale_ref[...] + shift_ref[...], 0.0)
    _emit_cols(h, col_ref, dims)
    _conv_tail(w_ref, y_ref, sum_ref, ssq_ref, col_ref)


def _bn_relu_kernel(scale_ref, shift_ref, x_ref, o_ref):
    o_ref[0] = jnp.maximum(x_ref[0] * scale_ref[...] + shift_ref[...], 0.0)


def _wmat(w):
    cout, cin = w.shape[:2]
    return jnp.transpose(w, (0, 2, 3, 4, 1)).reshape(cout, 27 * cin).astype(
        jnp.bfloat16)


def _bn_coeffs(psum, pssq, gamma, beta, count):
    total = jnp.sum(psum[:, :, 0], axis=0)
    total_sq = jnp.sum(pssq[:, :, 0], axis=0)
    mean = total / count
    var = total_sq / count - mean * mean
    scale = gamma * jax.lax.rsqrt(var + _EPS)
    shift = beta - mean * scale
    cout = gamma.shape[0]
    return scale.reshape(cout, 1), shift.reshape(cout, 1)


@jax.jit
def _down3d(x, w1, g1, be1, w2, g2, be2):
    n, cin, d, h, w = x.shape
    cout = w1.shape[0]
    d2, h2, w2s = d // 2, h // 2, w // 2
    m = d2 * h2 * w2s
    dims1 = (cin, d2, h2, w2s)
    dims2 = (cout, d2, h2, w2s)
    cp = pltpu.CompilerParams(dimension_semantics=("parallel",),
                              vmem_limit_bytes=_VMEM_LIMIT)

    views = jnp.stack(
        [x[:, :, i::2, j::2, k::2].reshape(n, cin, m)
         for i in range(2) for j in range(2) for k in range(2)], axis=0)

    y1, s1, q1 = pl.pallas_call(
        functools.partial(_pool_conv_kernel, dims=dims1),
        grid=(n,),
        in_specs=[pl.BlockSpec((8, 1, cin, m), lambda i: (0, i, 0, 0)),
                  pl.BlockSpec((cout, 27 * cin), lambda i: (0, 0))],
        out_specs=[pl.BlockSpec((1, cout, m), lambda i: (i, 0, 0)),
                   pl.BlockSpec((1, cout, 1), lambda i: (i, 0, 0)),
                   pl.BlockSpec((1, cout, 1), lambda i: (i, 0, 0))],
        out_shape=(jax.ShapeDtypeStruct((n, cout, m), jnp.float32),
                   jax.ShapeDtypeStruct((n, cout, 1), jnp.float32),
                   jax.ShapeDtypeStruct((n, cout, 1), jnp.float32)),
        scratch_shapes=[pltpu.VMEM((27 * cin, m), jnp.bfloat16)],
        compiler_params=cp,
    )(views, _wmat(w1))

    sc1, sh1 = _bn_coeffs(s1, q1, g1, be1, n * m)

    y2, s2, q2 = pl.pallas_call(
        functools.partial(_bn_conv_kernel, dims=dims2),
        grid=(n,),
        in_specs=[pl.BlockSpec((cout, 1), lambda i: (0, 0)),
                  pl.BlockSpec((cout, 1), lambda i: (0, 0)),
                  pl.BlockSpec((cout, 27 * cout), lambda i: (0, 0)),
                  pl.BlockSpec((1, cout, m), lambda i: (i, 0, 0))],
        out_specs=[pl.BlockSpec((1, cout, m), lambda i: (i, 0, 0)),
                   pl.BlockSpec((1, cout, 1), lambda i: (i, 0, 0)),
                   pl.BlockSpec((1, cout, 1), lambda i: (i, 0, 0))],
        out_shape=(jax.ShapeDtypeStruct((n, cout, m), jnp.float32),
                   jax.ShapeDtypeStruct((n, cout, 1), jnp.float32),
                   jax.ShapeDtypeStruct((n, cout, 1), jnp.float32)),
        scratch_shapes=[pltpu.VMEM((27 * cout, m), jnp.bfloat16)],
        compiler_params=cp,
    )(sc1, sh1, _wmat(w2), y1)

    sc2, sh2 = _bn_coeffs(s2, q2, g2, be2, n * m)

    out = pl.pallas_call(
        _bn_relu_kernel,
        grid=(n,),
        in_specs=[pl.BlockSpec((cout, 1), lambda i: (0, 0)),
                  pl.BlockSpec((cout, 1), lambda i: (0, 0)),
                  pl.BlockSpec((1, cout, m), lambda i: (i, 0, 0))],
        out_specs=pl.BlockSpec((1, cout, m), lambda i: (i, 0, 0)),
        out_shape=jax.ShapeDtypeStruct((n, cout, m), jnp.float32),
        compiler_params=cp,
    )(sc2, sh2, y2)

    return out.reshape(n, cout, d2, h2, w2s)


def kernel(x, w1, g1, be1, w2, g2, be2):
    return _down3d(x, w1, g1, be1, w2, g2, be2)


# speedup vs baseline: 110.6068x; 101.6490x over previous
import functools

import jax
import jax.numpy as jnp
from jax.experimental import pallas as pl
from jax.experimental.pallas import tpu as pltpu

_EPS = 1e-5
_VMEM_LIMIT = 48 * 2**20


def _round_up(x, n):
    return ((x + n - 1) // n) * n


def _emit_cols(a, col_ref, dims):
    c, d2, h2, w2 = dims
    m = d2 * h2 * w2
    sh, sd = w2, h2 * w2
    pad = sd + sh + 1
    zc = jnp.zeros((c, pad), a.dtype)
    ae = jnp.concatenate([zc, a, zc], axis=1)
    lane = jax.lax.broadcasted_iota(jnp.int32, (1, m), 1)
    ww = jax.lax.rem(lane, w2)
    hh = jax.lax.rem(lane // w2, h2)
    t = 0
    for kd in range(3):
        for kh in range(3):
            for kw in range(3):
                delta = (kd - 1) * sd + (kh - 1) * sh + (kw - 1)
                sl = jax.lax.slice(ae, (0, pad + delta), (c, pad + delta + m))
                conds = []
                if kh == 0:
                    conds.append(hh >= 1)
                elif kh == 2:
                    conds.append(hh < h2 - 1)
                if kw == 0:
                    conds.append(ww >= 1)
                elif kw == 2:
                    conds.append(ww < w2 - 1)
                if conds:
                    mask = functools.reduce(jnp.logical_and, conds)
                    sl = jnp.where(mask, sl, 0.0)
                col_ref[t * c:(t + 1) * c, :] = sl.astype(col_ref.dtype)
                t += 1


def _conv_tail(w_ref, y_ref, sum_ref, ssq_ref, col_ref):
    y = jnp.dot(w_ref[...], col_ref[...], preferred_element_type=jnp.float32)
    y_ref[0] = y
    sum_ref[0] = jnp.sum(y, axis=1, keepdims=True)
    ssq_ref[0] = jnp.sum(y * y, axis=1, keepdims=True)


def _maxpool(x_ref, pool_ref, dims):
    c, d2, h2, w2 = dims
    hw = 4 * h2 * w2
    sh = 2 * w2
    ml = h2 * w2
    ll = 2 * d2 * hw
    lmax = (h2 - 1) * 2 * sh + 2 * (w2 - 1)
    ks = _round_up(lmax + 2, 128)
    v = x_ref[0].astype(jnp.bfloat16)
    a = jnp.maximum(jax.lax.slice(v, (0, 0), (c, ll - hw)),
                    jax.lax.slice(v, (0, hw), (c, ll)))
    b = jnp.maximum(jax.lax.slice(a, (0, 0), (c, ll - hw - 1)),
                    jax.lax.slice(a, (0, 1), (c, ll - hw)))
    e = jnp.maximum(jax.lax.slice(b, (0, 0), (c, ll - hw - 1 - sh)),
                    jax.lax.slice(b, (0, sh), (c, ll - hw - 1)))
    ri = jax.lax.broadcasted_iota(jnp.int32, (ks, ml), 0)
    ci = jax.lax.broadcasted_iota(jnp.int32, (ks, ml), 1)
    sel = ((ci // w2) * (2 * sh) + jax.lax.rem(ci, w2) * 2 == ri)
    s = jnp.where(sel, 1.0, 0.0).astype(jnp.bfloat16)
    zs = jnp.zeros((c, ks - (lmax + 1)), jnp.bfloat16)
    for j in range(d2):
        cs = jax.lax.slice(e, (0, j * 2 * hw), (c, j * 2 * hw + lmax + 1))
        cs = jnp.concatenate([cs, zs], axis=1)
        pool_ref[:, j * ml:(j + 1) * ml] = jnp.dot(
            cs, s, preferred_element_type=jnp.float32).astype(jnp.bfloat16)


def _pool_conv_kernel(x_ref, w_ref, y_ref, sum_ref, ssq_ref, pool_ref,
                      col_ref, *, dims):
    _maxpool(x_ref, pool_ref, dims)
    _emit_cols(pool_ref[...], col_ref, dims)
    _conv_tail(w_ref, y_ref, sum_ref, ssq_ref, col_ref)


def _bn_conv_kernel(scale_ref, shift_ref, w_ref, x_ref, y_ref, sum_ref,
                    ssq_ref, col_ref, *, dims):
    h = jnp.maximum(x_ref[0] * scale_ref[...] + shift_ref[...], 0.0)
    _emit_cols(h, col_ref, dims)
    _conv_tail(w_ref, y_ref, sum_ref, ssq_ref, col_ref)


def _bn_relu_kernel(scale_ref, shift_ref, x_ref, o_ref):
    o_ref[0] = jnp.maximum(x_ref[0] * scale_ref[...] + shift_ref[...], 0.0)


def _wmat(w):
    cout, cin = w.shape[:2]
    return jnp.transpose(w, (0, 2, 3, 4, 1)).reshape(cout, 27 * cin).astype(
        jnp.bfloat16)


def _bn_coeffs(psum, pssq, gamma, beta, count):
    total = jnp.sum(psum[:, :, 0], axis=0)
    total_sq = jnp.sum(pssq[:, :, 0], axis=0)
    mean = total / count
    var = total_sq / count - mean * mean
    scale = gamma * jax.lax.rsqrt(var + _EPS)
    shift = beta - mean * scale
    cout = gamma.shape[0]
    return scale.reshape(cout, 1), shift.reshape(cout, 1)


@jax.jit
def _down3d(x, w1, g1, be1, w2, g2, be2):
    n, cin, d, h, w = x.shape
    cout = w1.shape[0]
    d2, h2, w2s = d // 2, h // 2, w // 2
    m = d2 * h2 * w2s
    dims1 = (cin, d2, h2, w2s)
    dims2 = (cout, d2, h2, w2s)
    cp = pltpu.CompilerParams(dimension_semantics=("parallel",),
                              vmem_limit_bytes=_VMEM_LIMIT)

    x4 = x.reshape(n, cin, d * h * w)

    y1, s1, q1 = pl.pallas_call(
        functools.partial(_pool_conv_kernel, dims=dims1),
        grid=(n,),
        in_specs=[pl.BlockSpec((1, cin, d * h * w), lambda i: (i, 0, 0)),
                  pl.BlockSpec((cout, 27 * cin), lambda i: (0, 0))],
        out_specs=[pl.BlockSpec((1, cout, m), lambda i: (i, 0, 0)),
                   pl.BlockSpec((1, cout, 1), lambda i: (i, 0, 0)),
                   pl.BlockSpec((1, cout, 1), lambda i: (i, 0, 0))],
        out_shape=(jax.ShapeDtypeStruct((n, cout, m), jnp.float32),
                   jax.ShapeDtypeStruct((n, cout, 1), jnp.float32),
                   jax.ShapeDtypeStruct((n, cout, 1), jnp.float32)),
        scratch_shapes=[pltpu.VMEM((cin, m), jnp.bfloat16),
                        pltpu.VMEM((27 * cin, m), jnp.bfloat16)],
        compiler_params=cp,
    )(x4, _wmat(w1))

    sc1, sh1 = _bn_coeffs(s1, q1, g1, be1, n * m)

    y2, s2, q2 = pl.pallas_call(
        functools.partial(_bn_conv_kernel, dims=dims2),
        grid=(n,),
        in_specs=[pl.BlockSpec((cout, 1), lambda i: (0, 0)),
                  pl.BlockSpec((cout, 1), lambda i: (0, 0)),
                  pl.BlockSpec((cout, 27 * cout), lambda i: (0, 0)),
                  pl.BlockSpec((1, cout, m), lambda i: (i, 0, 0))],
        out_specs=[pl.BlockSpec((1, cout, m), lambda i: (i, 0, 0)),
                   pl.BlockSpec((1, cout, 1), lambda i: (i, 0, 0)),
                   pl.BlockSpec((1, cout, 1), lambda i: (i, 0, 0))],
        out_shape=(jax.ShapeDtypeStruct((n, cout, m), jnp.float32),
                   jax.ShapeDtypeStruct((n, cout, 1), jnp.float32),
                   jax.ShapeDtypeStruct((n, cout, 1), jnp.float32)),
        scratch_shapes=[pltpu.VMEM((27 * cout, m), jnp.bfloat16)],
        compiler_params=cp,
    )(sc1, sh1, _wmat(w2), y1)

    sc2, sh2 = _bn_coeffs(s2, q2, g2, be2, n * m)

    out = pl.pallas_call(
        _bn_relu_kernel,
        grid=(n,),
        in_specs=[pl.BlockSpec((cout, 1), lambda i: (0, 0)),
                  pl.BlockSpec((cout, 1), lambda i: (0, 0)),
                  pl.BlockSpec((1, cout, m), lambda i: (i, 0, 0))],
        out_specs=pl.BlockSpec((1, cout, m), lambda i: (i, 0, 0)),
        out_shape=jax.ShapeDtypeStruct((n, cout, m), jnp.float32),
        compiler_params=cp,
    )(sc2, sh2, y2)

    return out.reshape(n, cout, d2, h2, w2s)


def kernel(x, w1, g1, be1, w2, g2, be2):
    return _down3d(x, w1, g1, be1, w2, g2, be2)


# speedup vs baseline: 113.6788x; 1.0278x over previous
import functools

import jax
import jax.numpy as jnp
from jax.experimental import pallas as pl
from jax.experimental.pallas import tpu as pltpu

_EPS = 1e-5
_VMEM_LIMIT = 48 * 2**20


def _round_up(x, n):
    return ((x + n - 1) // n) * n


def _emit_cols(a, col_ref, dims):
    c, d2, h2, w2 = dims
    m = d2 * h2 * w2
    sh, sd = w2, h2 * w2
    pad = sd + sh + 1
    zc = jnp.zeros((c, pad), a.dtype)
    ae = jnp.concatenate([zc, a, zc], axis=1)
    lane = jax.lax.broadcasted_iota(jnp.int32, (1, m), 1)
    ww = jax.lax.rem(lane, w2)
    hh = jax.lax.rem(lane // w2, h2)
    t = 0
    for kd in range(3):
        for kh in range(3):
            for kw in range(3):
                delta = (kd - 1) * sd + (kh - 1) * sh + (kw - 1)
                sl = jax.lax.slice(ae, (0, pad + delta), (c, pad + delta + m))
                conds = []
                if kh == 0:
                    conds.append(hh >= 1)
                elif kh == 2:
                    conds.append(hh < h2 - 1)
                if kw == 0:
                    conds.append(ww >= 1)
                elif kw == 2:
                    conds.append(ww < w2 - 1)
                if conds:
                    mask = functools.reduce(jnp.logical_and, conds)
                    sl = jnp.where(mask, sl, 0.0)
                col_ref[t * c:(t + 1) * c, :] = sl.astype(col_ref.dtype)
                t += 1


def _conv_tail(w_ref, y_ref, sum_ref, ssq_ref, col_ref):
    y = jnp.dot(w_ref[...], col_ref[...], preferred_element_type=jnp.float32)
    y_ref[0] = y.astype(y_ref.dtype)
    sum_ref[0] = jnp.sum(y, axis=1, keepdims=True)
    ssq_ref[0] = jnp.sum(y * y, axis=1, keepdims=True)


def _coeffs(sum_ref, ssq_ref, g_ref, be_ref, count):
    mean = jnp.sum(sum_ref[...], axis=0) / count
    var = jnp.sum(ssq_ref[...], axis=0) / count - mean * mean
    scale = g_ref[...] * jax.lax.rsqrt(var + _EPS)
    return scale, be_ref[...] - mean * scale


def _maxpool(x_ref, pool_ref, dims):
    c, d2, h2, w2 = dims
    hw = 4 * h2 * w2
    sh = 2 * w2
    ml = h2 * w2
    ll = 2 * d2 * hw
    lmax = (h2 - 1) * 2 * sh + 2 * (w2 - 1)
    ks = _round_up(lmax + 2, 128)
    v = x_ref[0].astype(jnp.bfloat16)
    a = jnp.maximum(jax.lax.slice(v, (0, 0), (c, ll - hw)),
                    jax.lax.slice(v, (0, hw), (c, ll)))
    b = jnp.maximum(jax.lax.slice(a, (0, 0), (c, ll - hw - 1)),
                    jax.lax.slice(a, (0, 1), (c, ll - hw)))
    e = jnp.maximum(jax.lax.slice(b, (0, 0), (c, ll - hw - 1 - sh)),
                    jax.lax.slice(b, (0, sh), (c, ll - hw - 1)))
    ri = jax.lax.broadcasted_iota(jnp.int32, (ks, ml), 0)
    ci = jax.lax.broadcasted_iota(jnp.int32, (ks, ml), 1)
    sel = ((ci // w2) * (2 * sh) + jax.lax.rem(ci, w2) * 2 == ri)
    s = jnp.where(sel, 1.0, 0.0).astype(jnp.bfloat16)
    zs = jnp.zeros((c, ks - (lmax + 1)), jnp.bfloat16)
    for j in range(d2):
        cs = jax.lax.slice(e, (0, j * 2 * hw), (c, j * 2 * hw + lmax + 1))
        cs = jnp.concatenate([cs, zs], axis=1)
        pool_ref[:, j * ml:(j + 1) * ml] = jnp.dot(
            cs, s, preferred_element_type=jnp.float32).astype(jnp.bfloat16)


def _pool_conv_kernel(x_ref, w_ref, y_ref, sum_ref, ssq_ref, pool_ref,
                      col_ref, *, dims):
    _maxpool(x_ref, pool_ref, dims)
    _emit_cols(pool_ref[...], col_ref, dims)
    _conv_tail(w_ref, y_ref, sum_ref, ssq_ref, col_ref)


def _bn_conv_kernel(psum_ref, pssq_ref, g_ref, be_ref, w_ref, x_ref, y_ref,
                    sum_ref, ssq_ref, col_ref, *, dims, count):
    scale, shift = _coeffs(psum_ref, pssq_ref, g_ref, be_ref, count)
    h = jnp.maximum(x_ref[0].astype(jnp.float32) * scale + shift, 0.0)
    _emit_cols(h, col_ref, dims)
    _conv_tail(w_ref, y_ref, sum_ref, ssq_ref, col_ref)


def _bn_relu_kernel(psum_ref, pssq_ref, g_ref, be_ref, x_ref, o_ref, *,
                    count):
    scale, shift = _coeffs(psum_ref, pssq_ref, g_ref, be_ref, count)
    o_ref[0] = jnp.maximum(x_ref[0].astype(jnp.float32) * scale + shift, 0.0)


def _wmat(w):
    cout, cin = w.shape[:2]
    return jnp.transpose(w, (0, 2, 3, 4, 1)).reshape(cout, 27 * cin).astype(
        jnp.bfloat16)


@jax.jit
def _down3d(x, w1, g1, be1, w2, g2, be2):
    n, cin, d, h, w = x.shape
    cout = w1.shape[0]
    d2, h2, w2s = d // 2, h // 2, w // 2
    m = d2 * h2 * w2s
    dims1 = (cin, d2, h2, w2s)
    dims2 = (cout, d2, h2, w2s)
    cp = pltpu.CompilerParams(dimension_semantics=("parallel",),
                              vmem_limit_bytes=_VMEM_LIMIT)

    x4 = x.reshape(n, cin, d * h * w)

    y1, s1, q1 = pl.pallas_call(
        functools.partial(_pool_conv_kernel, dims=dims1),
        grid=(n,),
        in_specs=[pl.BlockSpec((1, cin, d * h * w), lambda i: (i, 0, 0)),
                  pl.BlockSpec((cout, 27 * cin), lambda i: (0, 0))],
        out_specs=[pl.BlockSpec((1, cout, m), lambda i: (i, 0, 0)),
                   pl.BlockSpec((1, cout, 1), lambda i: (i, 0, 0)),
                   pl.BlockSpec((1, cout, 1), lambda i: (i, 0, 0))],
        out_shape=(jax.ShapeDtypeStruct((n, cout, m), jnp.bfloat16),
                   jax.ShapeDtypeStruct((n, cout, 1), jnp.float32),
                   jax.ShapeDtypeStruct((n, cout, 1), jnp.float32)),
        scratch_shapes=[pltpu.VMEM((cin, m), jnp.bfloat16),
                        pltpu.VMEM((27 * cin, m), jnp.bfloat16)],
        compiler_params=cp,
    )(x4, _wmat(w1))

    stat_specs = [pl.BlockSpec((n, cout, 1), lambda i: (0, 0, 0)),
                  pl.BlockSpec((n, cout, 1), lambda i: (0, 0, 0)),
                  pl.BlockSpec((cout, 1), lambda i: (0, 0)),
                  pl.BlockSpec((cout, 1), lambda i: (0, 0))]

    y2, s2, q2 = pl.pallas_call(
        functools.partial(_bn_conv_kernel, dims=dims2, count=n * m),
        grid=(n,),
        in_specs=stat_specs + [
            pl.BlockSpec((cout, 27 * cout), lambda i: (0, 0)),
            pl.BlockSpec((1, cout, m), lambda i: (i, 0, 0))],
        out_specs=[pl.BlockSpec((1, cout, m), lambda i: (i, 0, 0)),
                   pl.BlockSpec((1, cout, 1), lambda i: (i, 0, 0)),
                   pl.BlockSpec((1, cout, 1), lambda i: (i, 0, 0))],
        out_shape=(jax.ShapeDtypeStruct((n, cout, m), jnp.bfloat16),
                   jax.ShapeDtypeStruct((n, cout, 1), jnp.float32),
                   jax.ShapeDtypeStruct((n, cout, 1), jnp.float32)),
        scratch_shapes=[pltpu.VMEM((27 * cout, m), jnp.bfloat16)],
        compiler_params=cp,
    )(s1, q1, g1.reshape(cout, 1), be1.reshape(cout, 1), _wmat(w2), y1)

    out = pl.pallas_call(
        functools.partial(_bn_relu_kernel, count=n * m),
        grid=(n,),
        in_specs=stat_specs + [pl.BlockSpec((1, cout, m), lambda i: (i, 0, 0))],
        out_specs=pl.BlockSpec((1, cout, m), lambda i: (i, 0, 0)),
        out_shape=jax.ShapeDtypeStruct((n, cout, m), jnp.float32),
        compiler_params=cp,
    )(s2, q2, g2.reshape(cout, 1), be2.reshape(cout, 1), y2)

    return out.reshape(n, cout, d2, h2, w2s)


def kernel(x, w1, g1, be1, w2, g2, be2):
    return _down3d(x, w1, g1, be1, w2, g2, be2)
